# Initial kernel scaffold; baseline (speedup 1.0000x reference)
import jax
import jax.numpy as jnp
from jax.experimental import pallas as pl


def kernel(params, features, src, dst, edge_weight):
    raise NotImplementedError("write your pallas kernel here")


# speedup vs baseline: 4.3098x; 4.3098x over previous
import functools

import numpy as np

import jax
import jax.numpy as jnp
from jax import lax
from jax.experimental import pallas as pl
from jax.experimental.pallas import tpu as pltpu
from jax.experimental.pallas import tpu_sc as plsc

N = 10000
NP = 10112
D = 128
E2 = 640000
CHUNKS = 160
EPT = CHUNKS * 128
EPAD = EPT * 32
ROWS_PT = NP // 16
GB = 32
BN = 1000
BV = 400
NCB = 4096

f32 = jnp.float32
i32 = jnp.int32


def _sc_edge_body(y_hbm, src_hbm, ew_hbm, dst_hbm, out_hbm,
                  agg, sbuf, ebuf, dbuf, rows0, sem_g):
    c = lax.axis_index("c")
    s = lax.axis_index("s")
    wid = c * 16 + s
    zero16 = jnp.zeros((16,), f32)

    def zrow(r, carry):
        for k in range(8):
            rows0[r, pl.ds(k * 16, 16)] = zero16
        return carry
    lax.fori_loop(0, 128, zrow, 0)
    base = s * ROWS_PT
    for k in range(4):
        pltpu.sync_copy(rows0, agg.at[pl.ds(base + k * 128, 128), :])
    pltpu.sync_copy(rows0.at[pl.ds(0, ROWS_PT - 512), :],
                    agg.at[pl.ds(base + 512, ROWS_PT - 512), :])

    plsc.subcore_barrier()

    def group(g, carry):
        roff = wid * CHUNKS + g * GB
        pltpu.sync_copy(src_hbm.at[pl.ds(roff, GB), :], sbuf)
        pltpu.sync_copy(ew_hbm.at[pl.ds(roff, GB), :], ebuf)
        pltpu.sync_copy(dst_hbm.at[pl.ds(roff, GB), :], dbuf)

        def rcalc(j, c2):
            for k in range(8):
                e = ebuf[j, pl.ds(k * 16, 16)]
                sv = sbuf[j, pl.ds(k * 16, 16)]
                ec = jnp.where((e >= 1) & (e <= 4), e, 0)
                ebuf[j, pl.ds(k * 16, 16)] = ec * N + sv
            return c2
        lax.fori_loop(0, GB, rcalc, 0)

        def chunk(j, c2):
            pltpu.async_copy(y_hbm.at[ebuf.at[j]], rows0, sem_g).wait()
            pltpu.sync_copy(rows0, agg.at[dbuf.at[j]], add=True)
            return c2
        lax.fori_loop(0, GB, chunk, 0)
        return carry
    lax.fori_loop(0, CHUNKS // GB, group, 0)

    plsc.subcore_barrier()

    for k in range(4):
        pltpu.sync_copy(agg.at[pl.ds(base + k * 128, 128), :], rows0)
        pltpu.sync_copy(rows0, out_hbm.at[c, pl.ds(base + k * 128, 128), :])
    pltpu.sync_copy(agg.at[pl.ds(base + 512, ROWS_PT - 512), :],
                    rows0.at[pl.ds(0, ROWS_PT - 512), :])
    pltpu.sync_copy(rows0.at[pl.ds(0, ROWS_PT - 512), :],
                    out_hbm.at[c, pl.ds(base + 512, ROWS_PT - 512), :])


_sc_edge = pl.kernel(
    _sc_edge_body,
    out_type=jax.ShapeDtypeStruct((2, NP, D), f32),
    mesh=plsc.VectorSubcoreMesh(core_axis_name="c", subcore_axis_name="s"),
    scratch_types=[
        pltpu.VMEM_SHARED((NP, D), f32),
        pltpu.VMEM((GB, 128), i32),
        pltpu.VMEM((GB, 128), i32),
        pltpu.VMEM((GB, 128), i32),
        pltpu.VMEM((128, D), f32),
        pltpu.SemaphoreType.DMA,
    ],
)


def _embed_body(f_ref, off_ref, T_ref, s0T_ref, bond_ref, eW_ref, eb_ref,
                h0_ref, hs_ref, y_ref):
    adj = f_ref[...] + off_ref[...]
    iot = lax.broadcasted_iota(i32, (BN, 256), 1)
    oh = jnp.zeros((BN, 256), f32)
    for j in range(27):
        oh = oh + (adj[:, j][:, None] == iot).astype(f32)
    h0 = jnp.dot(oh, T_ref[...], preferred_element_type=f32)
    h0_ref[...] = h0
    hs_ref[...] = jnp.dot(h0, s0T_ref[...], preferred_element_type=f32)
    etab = jnp.dot(bond_ref[...], eW_ref[...],
                   preferred_element_type=f32) + eb_ref[...]
    for v in range(5):
        y_ref[v] = jnp.maximum(h0 + etab[v][None, :], 0.0)


def _node_body(with_skip, with_y, *refs):
    idx = 0
    x_ref = refs[idx]; idx += 1
    agg_ref = refs[idx]; idx += 1
    if with_skip:
        skip_ref = refs[idx]; idx += 1
    W1T_ref = refs[idx]; idx += 1
    b1_ref = refs[idx]; idx += 1
    W2Ts_ref = refs[idx]; idx += 1
    b2s_ref = refs[idx]; idx += 1
    g_ref = refs[idx]; idx += 1
    b_ref = refs[idx]; idx += 1
    if with_y:
        bond_ref = refs[idx]; idx += 1
        eW_ref = refs[idx]; idx += 1
        eb_ref = refs[idx]; idx += 1
    h_ref = refs[idx]; idx += 1
    if with_y:
        y_ref = refs[idx]; idx += 1

    x = x_ref[...]
    h = x + (agg_ref[0] + agg_ref[1])
    t = jnp.maximum(jnp.dot(h, W1T_ref[...], preferred_element_type=f32)
                    + b1_ref[...], 0.0)
    t2 = jnp.maximum(jnp.dot(t, W2Ts_ref[...], preferred_element_type=f32)
                     + b2s_ref[...], 0.0)
    z = t2 + (skip_ref[...] if with_skip else x)
    mu = jnp.mean(z, axis=1, keepdims=True)
    dfc = z - mu
    var = jnp.mean(dfc * dfc, axis=1, keepdims=True)
    hn = dfc / jnp.sqrt(var + 1e-5) * g_ref[...] + b_ref[...]
    h_ref[...] = hn
    if with_y:
        etab = jnp.dot(bond_ref[...], eW_ref[...],
                       preferred_element_type=f32) + eb_ref[...]
        for v in range(5):
            y_ref[v] = jnp.maximum(hn + etab[v][None, :], 0.0)


def _final_body(h0_ref, h1_ref, h2_ref, h3_ref,
                A0_ref, A1_ref, A2_ref, A3_ref, mb1_ref,
                W2T_ref, mb2_ref, oWT_ref, ob_ref,
                g_ref, b_ref, msk_ref, cbT_ref, cbp_ref, q_ref):
    u = (jnp.dot(h0_ref[...], A0_ref[...], preferred_element_type=f32)
         + jnp.dot(h1_ref[...], A1_ref[...], preferred_element_type=f32)
         + jnp.dot(h2_ref[...], A2_ref[...], preferred_element_type=f32)
         + jnp.dot(h3_ref[...], A3_ref[...], preferred_element_type=f32)
         + mb1_ref[...])
    t = jnp.maximum(u, 0.0)
    t2 = jnp.maximum(jnp.dot(t, W2T_ref[...], preferred_element_type=f32)
                     + mb2_ref[...], 0.0)
    ho = jnp.dot(t2, oWT_ref[...], preferred_element_type=f32) + ob_ref[...]
    mu = jnp.sum(ho, axis=1, keepdims=True) / 120.0
    dfc = (ho - mu) * msk_ref[...]
    var = jnp.sum(dfc * dfc, axis=1, keepdims=True) / 120.0
    hv = dfc / jnp.sqrt(var + 1e-5) * g_ref[...] + b_ref[...]
    hvsq = jnp.sum(hv * hv, axis=1, keepdims=True)
    sc = jnp.dot(hv, cbT_ref[...], preferred_element_type=f32)
    cbsq = jnp.sum(cbT_ref[...] * cbT_ref[...], axis=0, keepdims=True)
    d = hvsq - 2.0 * sc + cbsq
    m = jnp.min(d, axis=1, keepdims=True)
    iot = lax.broadcasted_iota(i32, (BV, NCB), 1)
    idx = jnp.min(jnp.where(d == m, iot, NCB), axis=1, keepdims=True)
    ohv = (iot == idx).astype(f32)
    q_ref[...] = jnp.dot(ohv, cbp_ref[...], preferred_element_type=f32)


def _full(shape):
    return pl.BlockSpec(shape, lambda i: tuple(0 for _ in shape))


def _embed_call(fpad, offs, T, s0T, bondp, eWT, eb):
    return pl.pallas_call(
        _embed_body,
        grid=(N // BN,),
        in_specs=[
            pl.BlockSpec((BN, 128), lambda i: (i, 0)),
            _full((1, 128)), _full((256, 128)), _full((128, 128)),
            _full((8, 32)), _full((32, 128)), _full((1, 128)),
        ],
        out_specs=[
            pl.BlockSpec((BN, D), lambda i: (i, 0)),
            pl.BlockSpec((BN, D), lambda i: (i, 0)),
            pl.BlockSpec((5, BN, D), lambda i: (0, i, 0)),
        ],
        out_shape=[
            jax.ShapeDtypeStruct((N, D), f32),
            jax.ShapeDtypeStruct((N, D), f32),
            jax.ShapeDtypeStruct((5, N, D), f32),
        ],
    )(fpad, offs, T, s0T, bondp, eWT, eb)


def _node_call(with_skip, with_y, x, agg, *args):
    in_specs = [pl.BlockSpec((BN, D), lambda i: (i, 0)),
                pl.BlockSpec((2, BN, D), lambda i: (0, i, 0))]
    if with_skip:
        in_specs.append(pl.BlockSpec((BN, D), lambda i: (i, 0)))
    in_specs += [_full((128, 128)), _full((1, 128)),
                 _full((128, 128)), _full((1, 128)),
                 _full((1, 128)), _full((1, 128))]
    if with_y:
        in_specs += [_full((8, 32)), _full((32, 128)), _full((1, 128))]
    out_specs = [pl.BlockSpec((BN, D), lambda i: (i, 0))]
    out_shape = [jax.ShapeDtypeStruct((N, D), f32)]
    if with_y:
        out_specs.append(pl.BlockSpec((5, BN, D), lambda i: (0, i, 0)))
        out_shape.append(jax.ShapeDtypeStruct((5, N, D), f32))
    res = pl.pallas_call(
        functools.partial(_node_body, with_skip, with_y),
        grid=(N // BN,),
        in_specs=in_specs,
        out_specs=out_specs,
        out_shape=out_shape,
    )(x, agg, *args)
    return res if with_y else res[0]


def _final_call(h0, h1, h2, h3, *args):
    in_specs = [pl.BlockSpec((BV, D), lambda i: (i, 0)) for _ in range(4)]
    in_specs += [_full((128, 256)) for _ in range(4)]
    in_specs += [_full((1, 256)), _full((256, 128)), _full((1, 128)),
                 _full((128, 128)), _full((1, 128)),
                 _full((1, 128)), _full((1, 128)), _full((1, 128)),
                 _full((128, NCB)), _full((NCB, 128))]
    return pl.pallas_call(
        _final_body,
        grid=(N // BV,),
        in_specs=in_specs,
        out_specs=pl.BlockSpec((BV, D), lambda i: (i, 0)),
        out_shape=jax.ShapeDtypeStruct((N, D), f32),
    )(h0, h1, h2, h3, *args)


_OFFS = np.full((1, 128), -(1 << 20), np.int32)
_OFFS[0, :7] = [0, 120, 127 + 1, 134, 142, 144, 150]
for _i in range(20):
    _OFFS[0, 7 + _i] = 155 + 2 * _i


def _pad2(a, rows, cols):
    return jnp.pad(a, ((0, rows - a.shape[0]), (0, cols - a.shape[1])))


def _row(a, cols=128):
    return jnp.pad(a, (0, cols - a.shape[0]))[None, :]


def kernel(params, features, src, dst, edge_weight):
    p = params

    T = jnp.zeros((256, 128), f32)
    T = T.at[0:120, 0:16].set(p['elem'])
    T = T.at[120:127, 16:20].set(p['degree'])
    T = T.at[127:134, 20:24].set(p['valence'])
    T = T.at[134:142, 24:28].set(p['charge'])
    T = T.at[142:144, 28:32].set(p['aromatic'])
    T = T.at[144:150, 32:36].set(p['hybrid'])
    T = T.at[150:155, 36:40].set(p['hydrogen'])
    for i in range(20):
        T = T.at[155 + 2 * i:157 + 2 * i, 40 + 4 * i:44 + 4 * i].set(p['bin'][i])

    fpad = jnp.pad(features.astype(i32), ((0, 0), (0, 128 - 27)))
    offs = jnp.asarray(_OFFS)
    s0T = _pad2(p['skip0'].T, 128, 128)
    bondp = jnp.pad(p['bond'], ((0, 3), (0, 0)))

    eWT = {}
    ebp = {}
    W1T = {}
    b1p = {}
    W2Ts = {}
    b2s = {}
    for l, r in (('g1', 'res1'), ('g2', 'res2'), ('g3', 'res3')):
        eWT[l] = _pad2(p[l + '_eW'].T, 32, 128)
        ebp[l] = _row(p[l + '_eb'])
        W1T[l] = _pad2(p[l + '_W1'].T, 128, 128)
        b1p[l] = _row(p[l + '_b1'])
        W2Ts[l] = _pad2(p[l + '_W2'].T * p[r], 128, 128)
        b2s[l] = _row(p[l + '_b2'] * p[r])

    lng = {l: _row(p[l + '_g']) for l in ('ln1', 'ln2', 'ln3')}
    lnb = {l: _row(p[l + '_b']) for l in ('ln1', 'ln2', 'ln3')}

    A0 = _pad2(p['mix_W1'][:, 0:120].T, 128, 256)
    A1 = p['mix_W1'][:, 120:248].T
    A2 = p['mix_W1'][:, 248:376].T
    A3 = p['mix_W1'][:, 376:504].T
    mb1 = p['mix_b1'][None, :]
    mW2T = p['mix_W2'].T
    mb2 = _row(p['mix_b2'])
    oWT = _pad2(p['out_W'].T, 128, 128)
    obp = _row(p['out_b'])
    lnvg = _row(p['lnvq_g'])
    lnvb = _row(p['lnvq_b'])
    msk = jnp.asarray(np.concatenate([np.ones((1, 120), np.float32),
                                      np.zeros((1, 8), np.float32)], axis=1))
    cbT = _pad2(p['cb'].T, 128, NCB)
    cbp = _pad2(p['cb'], NCB, 128)

    s2 = jnp.concatenate([src, dst]).astype(i32)
    d2 = jnp.concatenate([dst, src]).astype(i32)
    e2 = jnp.concatenate([edge_weight, edge_weight]).astype(i32)
    s2 = jnp.pad(s2, (0, EPAD - E2)).reshape(EPAD // 128, 128)
    d2 = jnp.pad(d2, (0, EPAD - E2),
                 constant_values=N).reshape(EPAD // 128, 128)
    e2 = jnp.pad(e2, (0, EPAD - E2)).reshape(EPAD // 128, 128)

    h0, h0skip, y1 = _embed_call(fpad, offs, T, s0T, bondp, eWT['g1'],
                                 ebp['g1'])
    agg1 = _sc_edge(y1.reshape(5 * N, D), s2, e2, d2)
    h1, y2 = _node_call(True, True, h0, agg1, h0skip,
                        W1T['g1'], b1p['g1'], W2Ts['g1'], b2s['g1'],
                        lng['ln1'], lnb['ln1'], bondp, eWT['g2'], ebp['g2'])
    agg2 = _sc_edge(y2.reshape(5 * N, D), s2, e2, d2)
    h2, y3 = _node_call(False, True, h1, agg2,
                        W1T['g2'], b1p['g2'], W2Ts['g2'], b2s['g2'],
                        lng['ln2'], lnb['ln2'], bondp, eWT['g3'], ebp['g3'])
    agg3 = _sc_edge(y3.reshape(5 * N, D), s2, e2, d2)
    h3 = _node_call(False, False, h2, agg3,
                    W1T['g3'], b1p['g3'], W2Ts['g3'], b2s['g3'],
                    lng['ln3'], lnb['ln3'])
    q = _final_call(h0, h1, h2, h3, A0, A1, A2, A3, mb1, mW2T, mb2,
                    oWT, obp, lnvg, lnvb, msk, cbT, cbp)
    return q[:, :120]
